# Initial kernel scaffold; baseline (speedup 1.0000x reference)
import jax
import jax.numpy as jnp
from jax.experimental import pallas as pl


def kernel(x, edge_index, batch, W1, b1, W2, b2, W3, b3, att1, att2, gamma, beta, lw1, lb1, lw2, lb2, lw3, lb3):
    raise NotImplementedError("write your pallas kernel here")


# speedup vs baseline: 27.7983x; 27.7983x over previous
import functools
import jax
import jax.numpy as jnp
from jax import lax
from jax.experimental import pallas as pl
from jax.experimental.pallas import tpu as pltpu
from jax.experimental.pallas import tpu_sc as plsc

G, P, D, NH, NCLS = 50, 200, 128, 128, 10
EPG = 6400
LAMB = 1.0
K1 = 100
K2 = 50
N = G * P
E = G * EPG
NPF = N * P


_SC_CORES = 2
_SC_TILES = 16
_NW = _SC_CORES * _SC_TILES
_G2 = G - _NW
_PP = P * P


def _sc_at_body(src_hbm, dst_hbm, zero_hbm, out_hbm, src_v, dst_v, a_v):
    c = lax.axis_index("c")
    s = lax.axis_index("s")
    wid = s * _SC_CORES + c
    two = wid < _G2
    g0 = jnp.where(two, 2 * wid, _G2 + wid)

    ones = jnp.full((16,), 1.0, jnp.float32)

    for i in range(2):
        @pl.when((i == 0) | two)
        def _():
            g = g0 + i
            pltpu.sync_copy(zero_hbm.at[pl.ds(0, _PP)], a_v.at[pl.ds(0, _PP)])
            ebase = g * EPG
            pltpu.sync_copy(src_hbm.at[pl.ds(ebase, EPG)],
                            src_v.at[pl.ds(0, EPG)])
            pltpu.sync_copy(dst_hbm.at[pl.ds(ebase, EPG)],
                            dst_v.at[pl.ds(0, EPG)])

            def _chunk(j, carry):
                sv = src_v[pl.ds(j * 16, 16)]
                dv = dst_v[pl.ds(j * 16, 16)]
                fi = lax.rem(dv, P) * P + lax.rem(sv, P)
                plsc.addupdate_scatter(a_v, [fi], ones)
                return carry

            lax.fori_loop(0, EPG // 16, _chunk, 0)

            pltpu.sync_copy(a_v.at[pl.ds(0, _PP)],
                            out_hbm.at[pl.ds(g * _PP, _PP)])


def _sc_build_at(src, dst, zeros_init):
    mesh = plsc.VectorSubcoreMesh(core_axis_name="c", subcore_axis_name="s")
    run = functools.partial(
        pl.kernel,
        mesh=mesh,
        compiler_params=pltpu.CompilerParams(needs_layout_passes=False),
        out_type=jax.ShapeDtypeStruct((NPF,), jnp.float32),
        scratch_types=[
            pltpu.VMEM((EPG,), jnp.int32),
            pltpu.VMEM((EPG,), jnp.int32),
            pltpu.VMEM((_PP,), jnp.float32),
        ],
    )(_sc_at_body)
    return run(src, dst, zeros_init)


def _t_col_to_row(colv, n):
    eye = (lax.broadcasted_iota(jnp.int32, (n, n), 0)
           == lax.broadcasted_iota(jnp.int32, (n, n), 1)).astype(colv.dtype)
    return jnp.sum(eye * colv, axis=0, keepdims=True)


def _perm_matrix(score_col, n, k):
    s_row = _t_col_to_row(score_col, n)
    gt = (s_row > score_col).astype(jnp.float32)
    jlt = (lax.broadcasted_iota(jnp.int32, (n, n), 1)
           < lax.broadcasted_iota(jnp.int32, (n, n), 0))
    eqm = ((s_row == score_col) & jlt).astype(jnp.float32)
    rank = jnp.sum(gt + eqm, axis=1, keepdims=True)
    rank_row = _t_col_to_row(rank, n)
    kio = lax.broadcasted_iota(jnp.int32, (k, n), 0).astype(jnp.float32)
    pm = (kio == rank_row).astype(jnp.float32)
    sv = jnp.sum(pm * s_row, axis=1, keepdims=True)
    return pm, sv


def _mm(a, b):
    return jnp.dot(a, b, preferred_element_type=jnp.float32,
                   precision=lax.Precision.HIGHEST)


def _mm_nt(a, b):
    return lax.dot_general(a, b, (((1,), (1,)), ((), ())),
                           preferred_element_type=jnp.float32,
                           precision=lax.Precision.HIGHEST)


def _mm_bf(a, b):
    return jnp.dot(a.astype(jnp.bfloat16), b.astype(jnp.bfloat16),
                   preferred_element_type=jnp.float32)


def _mm_nt_bf(a, b):
    return lax.dot_general(a.astype(jnp.bfloat16), b.astype(jnp.bfloat16),
                           (((1,), (1,)), ((), ())),
                           preferred_element_type=jnp.float32)


def _k1_body(at_ref, x_ref, w1_ref, b1_ref, a1a_ref, a1b_ref,
             x1_ref, xs_ref, adj1_ref, r1_ref, bnsum_ref, bnsq_ref):
    g = pl.program_id(0)
    at = at_ref[0]
    x = x_ref[0]

    indeg = jnp.sum(at, axis=1, keepdims=True)
    deg = indeg + 1.0
    dinv = lax.rsqrt(deg)
    h = dinv * _mm(at, x * dinv) + dinv * dinv * x
    x1 = jnp.maximum(_mm_bf(h, w1_ref[...]) + b1_ref[...], 0.0)
    x1_ref[0] = x1

    degn = jnp.maximum(indeg, 1.0)
    neigh = _mm(at, x1) / degn
    score = jnp.sum(jnp.abs(x1 - neigh), axis=1, keepdims=True)

    pm, sv = _perm_matrix(score, P, K1)
    xs = _mm(pm, x1) * jnp.tanh(sv)
    xs_ref[0] = xs

    u = _mm_nt(pm, at)
    asel = _mm_nt(u, pm)

    s_i = _mm_bf(xs, a1a_ref[...])
    s_j = _mm_nt_bf(a1b_ref[...], xs)
    e1 = jnp.maximum(s_i + s_j, 0.0) + LAMB * asel
    m = jnp.max(e1, axis=1, keepdims=True)
    ex = jnp.exp(e1 - m)
    adj1 = ex / jnp.sum(ex, axis=1, keepdims=True)
    adj1_ref[0] = adj1

    r1_ref[0, :, 0:NH] = jnp.max(xs, axis=0, keepdims=True)
    r1_ref[0, :, NH:2 * NH] = jnp.mean(xs, axis=0, keepdims=True)

    @pl.when(g == 0)
    def _():
        bnsum_ref[...] = jnp.zeros_like(bnsum_ref)
        bnsq_ref[...] = jnp.zeros_like(bnsq_ref)

    bnsum_ref[...] += jnp.sum(xs, axis=0, keepdims=True)
    bnsq_ref[...] += jnp.sum(xs * xs, axis=0, keepdims=True)


def _k2_body(xs_ref, adj1_ref, r1_ref, bnsum_ref, bnsq_ref, gamma_ref,
             beta_ref, w2_ref, b2_ref, a2a_ref, a2b_ref, w3_ref, b3_ref,
             hh_ref):
    xs = xs_ref[0]
    adj1 = adj1_ref[0]
    inv_n = 1.0 / (G * K1)
    mu = bnsum_ref[...] * inv_n
    var = bnsq_ref[...] * inv_n - mu * mu
    xb = (xs - mu) * lax.rsqrt(var + 1e-5) * gamma_ref[...] + beta_ref[...]

    x2 = jnp.maximum(_mm_bf(_mm_bf(adj1, xb), w2_ref[...]) + b2_ref[...], 0.0)
    neigh2 = _mm_bf(adj1, x2)
    score2 = jnp.sum(jnp.abs(x2 - neigh2), axis=1, keepdims=True)

    pm2, sv2 = _perm_matrix(score2, K1, K2)
    xs2 = _mm(pm2, x2) * jnp.tanh(sv2)

    u2 = _mm(pm2, adj1)
    a1sel = _mm_nt(u2, pm2)

    t_i = _mm_bf(xs2, a2a_ref[...])
    t_j = _mm_nt_bf(a2b_ref[...], xs2)
    e2 = jnp.maximum(t_i + t_j, 0.0) + LAMB * a1sel
    m2 = jnp.max(e2, axis=1, keepdims=True)
    ex2 = jnp.exp(e2 - m2)
    adj2 = ex2 / jnp.sum(ex2, axis=1, keepdims=True)

    x3 = jnp.maximum(_mm_bf(_mm_bf(adj2, xs2), w3_ref[...]) + b3_ref[...], 0.0)

    r2 = jnp.concatenate([jnp.max(xs2, axis=0, keepdims=True),
                          jnp.mean(xs2, axis=0, keepdims=True)], axis=1)
    r3 = jnp.concatenate([jnp.max(x3, axis=0, keepdims=True),
                          jnp.mean(x3, axis=0, keepdims=True)], axis=1)
    hh_ref[0] = (jnp.maximum(r1_ref[0], 0.0) + jnp.maximum(r2, 0.0)
                 + jnp.maximum(r3, 0.0))


def _k3_body(hh_ref, lw1_ref, lb1_ref, lw2_ref, lb2_ref, lw3_ref, lb3_ref,
             x_ref, out_ref):
    hh = jnp.maximum(_mm_bf(hh_ref[...], lw1_ref[...]) + lb1_ref[...], 0.0)
    xo = jnp.maximum(_mm_bf(hh, lw2_ref[...]) + lb2_ref[...], 0.0)
    x_ref[...] = xo
    lg = _mm_bf(xo, lw3_ref[...]) + lb3_ref[...]
    mx = jnp.max(lg, axis=1, keepdims=True)
    z = lg - mx
    out_ref[...] = z - jnp.log(jnp.sum(jnp.exp(z), axis=1, keepdims=True))


def _run_k1(at4, x3d, W1, b1, a1a, a1b):
    fp = jnp.float32
    return pl.pallas_call(
        _k1_body,
        grid=(G,),
        in_specs=[
            pl.BlockSpec((1, P, P), lambda g: (g, 0, 0)),
            pl.BlockSpec((1, P, D), lambda g: (g, 0, 0)),
            pl.BlockSpec((D, NH), lambda g: (0, 0)),
            pl.BlockSpec((1, NH), lambda g: (0, 0)),
            pl.BlockSpec((NH, 1), lambda g: (0, 0)),
            pl.BlockSpec((1, NH), lambda g: (0, 0)),
        ],
        out_specs=[
            pl.BlockSpec((1, P, NH), lambda g: (g, 0, 0)),
            pl.BlockSpec((1, K1, NH), lambda g: (g, 0, 0)),
            pl.BlockSpec((1, K1, K1), lambda g: (g, 0, 0)),
            pl.BlockSpec((1, 1, 2 * NH), lambda g: (g, 0, 0)),
            pl.BlockSpec((1, NH), lambda g: (0, 0)),
            pl.BlockSpec((1, NH), lambda g: (0, 0)),
        ],
        out_shape=[
            jax.ShapeDtypeStruct((G, P, NH), fp),
            jax.ShapeDtypeStruct((G, K1, NH), fp),
            jax.ShapeDtypeStruct((G, K1, K1), fp),
            jax.ShapeDtypeStruct((G, 1, 2 * NH), fp),
            jax.ShapeDtypeStruct((1, NH), fp),
            jax.ShapeDtypeStruct((1, NH), fp),
        ],
    )(at4, x3d, W1, b1, a1a, a1b)


def _run_k2(xs, adj1, r1, bnsum, bnsq, gamma, beta, W2, b2, a2a, a2b, W3, b3):
    fp = jnp.float32
    return pl.pallas_call(
        _k2_body,
        grid=(G,),
        in_specs=[
            pl.BlockSpec((1, K1, NH), lambda g: (g, 0, 0)),
            pl.BlockSpec((1, K1, K1), lambda g: (g, 0, 0)),
            pl.BlockSpec((1, 1, 2 * NH), lambda g: (g, 0, 0)),
            pl.BlockSpec((1, NH), lambda g: (0, 0)),
            pl.BlockSpec((1, NH), lambda g: (0, 0)),
            pl.BlockSpec((1, NH), lambda g: (0, 0)),
            pl.BlockSpec((1, NH), lambda g: (0, 0)),
            pl.BlockSpec((NH, NH), lambda g: (0, 0)),
            pl.BlockSpec((1, NH), lambda g: (0, 0)),
            pl.BlockSpec((NH, 1), lambda g: (0, 0)),
            pl.BlockSpec((1, NH), lambda g: (0, 0)),
            pl.BlockSpec((NH, NH), lambda g: (0, 0)),
            pl.BlockSpec((1, NH), lambda g: (0, 0)),
        ],
        out_specs=[pl.BlockSpec((1, 1, 2 * NH), lambda g: (g, 0, 0))],
        out_shape=[jax.ShapeDtypeStruct((G, 1, 2 * NH), fp)],
    )(xs, adj1, r1, bnsum, bnsq, gamma, beta, W2, b2, a2a, a2b, W3, b3)[0]


def _run_k3(hh, lw1, lb1, lw2, lb2, lw3, lb3):
    fp = jnp.float32
    return pl.pallas_call(
        _k3_body,
        out_shape=[
            jax.ShapeDtypeStruct((G, NH // 2), fp),
            jax.ShapeDtypeStruct((G, NCLS), fp),
        ],
    )(hh, lw1, lb1, lw2, lb2, lw3, lb3)


def kernel(x, edge_index, batch, W1, b1, W2, b2, W3, b3, att1, att2,
           gamma, beta, lw1, lb1, lw2, lb2, lw3, lb3):
    del batch
    fp = jnp.float32
    src = edge_index[0]
    dst = edge_index[1]
    zeros_init = jnp.zeros((_PP,), fp)

    at_flat = _sc_build_at(src, dst, zeros_init)
    at4 = at_flat.reshape(G, P, P)

    x3d = x.reshape(G, P, D)
    b1r = b1.reshape(1, NH)
    a1a = att1[:NH].reshape(NH, 1)
    a1b = att1[NH:].reshape(1, NH)
    a2a = att2[:NH].reshape(NH, 1)
    a2b = att2[NH:].reshape(1, NH)

    x1, xs, adj1, r1, bnsum, bnsq = _run_k1(at4, x3d, W1, b1r, a1a, a1b)
    hh = _run_k2(xs, adj1, r1, bnsum, bnsq, gamma.reshape(1, NH),
                 beta.reshape(1, NH), W2, b2.reshape(1, NH), a2a, a2b,
                 W3, b3.reshape(1, NH))
    x_, out = _run_k3(hh.reshape(G, 2 * NH), lw1, lb1.reshape(1, NH), lw2,
                      lb2.reshape(1, NH // 2), lw3, lb3.reshape(1, NCLS))
    return (x_, out, x1.reshape(N, NH))


# speedup vs baseline: 31.3322x; 1.1271x over previous
import functools
import jax
import jax.numpy as jnp
from jax import lax
from jax.experimental import pallas as pl
from jax.experimental.pallas import tpu as pltpu
from jax.experimental.pallas import tpu_sc as plsc

G, P, D, NH, NCLS = 50, 200, 128, 128, 10
EPG = 6400
LAMB = 1.0
K1 = 100
K2 = 50
N = G * P
E = G * EPG
NPF = N * P


_SC_CORES = 2
_SC_TILES = 16
_NW = _SC_CORES * _SC_TILES
_G2 = G - _NW
_PP = P * P


def _sc_at_body(src_hbm, dst_hbm, zero_hbm, out_hbm, src_v, dst_v, a_v):
    c = lax.axis_index("c")
    s = lax.axis_index("s")
    wid = s * _SC_CORES + c
    two = wid < _G2
    g0 = jnp.where(two, 2 * wid, _G2 + wid)

    ones = jnp.full((16,), 1.0, jnp.float32)

    for i in range(2):
        @pl.when((i == 0) | two)
        def _():
            g = g0 + i
            pltpu.sync_copy(zero_hbm.at[pl.ds(0, _PP)], a_v.at[pl.ds(0, _PP)])
            ebase = g * EPG
            pltpu.sync_copy(src_hbm.at[pl.ds(ebase, EPG)],
                            src_v.at[pl.ds(0, EPG)])
            pltpu.sync_copy(dst_hbm.at[pl.ds(ebase, EPG)],
                            dst_v.at[pl.ds(0, EPG)])

            def _chunk(j, carry):
                sv = src_v[pl.ds(j * 16, 16)]
                dv = dst_v[pl.ds(j * 16, 16)]
                fi = lax.rem(dv, P) * P + lax.rem(sv, P)
                plsc.addupdate_scatter(a_v, [fi], ones)
                return carry

            lax.fori_loop(0, EPG // 16, _chunk, 0)

            pltpu.sync_copy(a_v.at[pl.ds(0, _PP)],
                            out_hbm.at[pl.ds(g * _PP, _PP)])


def _sc_build_at(src, dst, zeros_init):
    mesh = plsc.VectorSubcoreMesh(core_axis_name="c", subcore_axis_name="s")
    run = functools.partial(
        pl.kernel,
        mesh=mesh,
        compiler_params=pltpu.CompilerParams(needs_layout_passes=False),
        out_type=jax.ShapeDtypeStruct((NPF,), jnp.float32),
        scratch_types=[
            pltpu.VMEM((EPG,), jnp.int32),
            pltpu.VMEM((EPG,), jnp.int32),
            pltpu.VMEM((_PP,), jnp.float32),
        ],
    )(_sc_at_body)
    return run(src, dst, zeros_init)


def _t_col_to_row(colv, n):
    eye = (lax.broadcasted_iota(jnp.int32, (n, n), 0)
           == lax.broadcasted_iota(jnp.int32, (n, n), 1)).astype(colv.dtype)
    return jnp.sum(eye * colv, axis=0, keepdims=True)


def _perm_matrix(score_col, n, k):
    s_row = _t_col_to_row(score_col, n)
    gt = (s_row > score_col).astype(jnp.float32)
    jlt = (lax.broadcasted_iota(jnp.int32, (n, n), 1)
           < lax.broadcasted_iota(jnp.int32, (n, n), 0))
    eqm = ((s_row == score_col) & jlt).astype(jnp.float32)
    rank = jnp.sum(gt + eqm, axis=1, keepdims=True)
    rank_row = _t_col_to_row(rank, n)
    kio = lax.broadcasted_iota(jnp.int32, (k, n), 0).astype(jnp.float32)
    pm = (kio == rank_row).astype(jnp.float32)
    sv = jnp.sum(pm * s_row, axis=1, keepdims=True)
    return pm, sv


def _dot(a, b):
    return jnp.dot(a, b, preferred_element_type=jnp.float32)


def _dot_nt(a, b):
    return lax.dot_general(a, b, (((1,), (1,)), ((), ())),
                           preferred_element_type=jnp.float32)


def _split3(x):
    b0 = x.astype(jnp.bfloat16)
    r = x - b0.astype(jnp.float32)
    b1 = r.astype(jnp.bfloat16)
    b2 = (r - b1.astype(jnp.float32)).astype(jnp.bfloat16)
    return (b0, b1, b2)


def _mm3s(a_ex, bs):
    ab = a_ex.astype(jnp.bfloat16)
    return (_dot(ab, bs[0]) + _dot(ab, bs[1])) + _dot(ab, bs[2])


def _mm3(a_ex, b):
    return _mm3s(a_ex, _split3(b))


def _mm3_vt(a, b_ex):
    a0, a1, a2 = _split3(a)
    bb = b_ex.astype(jnp.bfloat16)
    return (_dot_nt(a0, bb) + _dot_nt(a1, bb)) + _dot_nt(a2, bb)


def _mm1_nt(a_ex, b_ex):
    return _dot_nt(a_ex.astype(jnp.bfloat16), b_ex.astype(jnp.bfloat16))


def _mm_bf(a, b):
    return jnp.dot(a.astype(jnp.bfloat16), b.astype(jnp.bfloat16),
                   preferred_element_type=jnp.float32)


def _mm_nt_bf(a, b):
    return lax.dot_general(a.astype(jnp.bfloat16), b.astype(jnp.bfloat16),
                           (((1,), (1,)), ((), ())),
                           preferred_element_type=jnp.float32)


def _k1_body(at_ref, x_ref, w1_ref, b1_ref, a1a_ref, a1b_ref,
             x1_ref, xs_ref, adj1_ref, r1_ref, bnsum_ref, bnsq_ref):
    g = pl.program_id(0)
    at = at_ref[0]
    x = x_ref[0]

    indeg = jnp.sum(at, axis=1, keepdims=True)
    deg = indeg + 1.0
    dinv = lax.rsqrt(deg)
    h = dinv * _mm3(at, x * dinv) + dinv * dinv * x
    x1 = jnp.maximum(_mm_bf(h, w1_ref[...]) + b1_ref[...], 0.0)
    x1_ref[0] = x1

    degn = jnp.maximum(indeg, 1.0)
    x1s = _split3(x1)
    neigh = _mm3s(at, x1s) / degn
    score = jnp.sum(jnp.abs(x1 - neigh), axis=1, keepdims=True)

    pm, sv = _perm_matrix(score, P, K1)
    xs = _mm3s(pm, x1s) * jnp.tanh(sv)
    xs_ref[0] = xs

    u = _mm1_nt(pm, at)
    asel = _mm1_nt(u, pm)

    s_i = _mm_bf(xs, a1a_ref[...])
    s_j = _mm_nt_bf(a1b_ref[...], xs)
    e1 = jnp.maximum(s_i + s_j, 0.0) + LAMB * asel
    m = jnp.max(e1, axis=1, keepdims=True)
    ex = jnp.exp(e1 - m)
    adj1 = ex / jnp.sum(ex, axis=1, keepdims=True)
    adj1_ref[0] = adj1

    r1_ref[0, :, 0:NH] = jnp.max(xs, axis=0, keepdims=True)
    r1_ref[0, :, NH:2 * NH] = jnp.mean(xs, axis=0, keepdims=True)

    @pl.when(g == 0)
    def _():
        bnsum_ref[...] = jnp.zeros_like(bnsum_ref)
        bnsq_ref[...] = jnp.zeros_like(bnsq_ref)

    bnsum_ref[...] += jnp.sum(xs, axis=0, keepdims=True)
    bnsq_ref[...] += jnp.sum(xs * xs, axis=0, keepdims=True)


def _k2_body(xs_ref, adj1_ref, r1_ref, bnsum_ref, bnsq_ref, gamma_ref,
             beta_ref, w2_ref, b2_ref, a2a_ref, a2b_ref, w3_ref, b3_ref,
             hh_ref):
    xs = xs_ref[0]
    adj1 = adj1_ref[0]
    inv_n = 1.0 / (G * K1)
    mu = bnsum_ref[...] * inv_n
    var = bnsq_ref[...] * inv_n - mu * mu
    xb = (xs - mu) * lax.rsqrt(var + 1e-5) * gamma_ref[...] + beta_ref[...]

    x2 = jnp.maximum(_mm_bf(_mm_bf(adj1, xb), w2_ref[...]) + b2_ref[...], 0.0)
    neigh2 = _mm_bf(adj1, x2)
    score2 = jnp.sum(jnp.abs(x2 - neigh2), axis=1, keepdims=True)

    pm2, sv2 = _perm_matrix(score2, K1, K2)
    xs2 = _mm3(pm2, x2) * jnp.tanh(sv2)

    u2 = _mm3(pm2, adj1)
    a1sel = _mm3_vt(u2, pm2)

    t_i = _mm_bf(xs2, a2a_ref[...])
    t_j = _mm_nt_bf(a2b_ref[...], xs2)
    e2 = jnp.maximum(t_i + t_j, 0.0) + LAMB * a1sel
    m2 = jnp.max(e2, axis=1, keepdims=True)
    ex2 = jnp.exp(e2 - m2)
    adj2 = ex2 / jnp.sum(ex2, axis=1, keepdims=True)

    x3 = jnp.maximum(_mm_bf(_mm_bf(adj2, xs2), w3_ref[...]) + b3_ref[...], 0.0)

    r2 = jnp.concatenate([jnp.max(xs2, axis=0, keepdims=True),
                          jnp.mean(xs2, axis=0, keepdims=True)], axis=1)
    r3 = jnp.concatenate([jnp.max(x3, axis=0, keepdims=True),
                          jnp.mean(x3, axis=0, keepdims=True)], axis=1)
    hh_ref[0] = (jnp.maximum(r1_ref[0], 0.0) + jnp.maximum(r2, 0.0)
                 + jnp.maximum(r3, 0.0))


def _k3_body(hh_ref, lw1_ref, lb1_ref, lw2_ref, lb2_ref, lw3_ref, lb3_ref,
             x_ref, out_ref):
    hh = jnp.maximum(_mm_bf(hh_ref[...], lw1_ref[...]) + lb1_ref[...], 0.0)
    xo = jnp.maximum(_mm_bf(hh, lw2_ref[...]) + lb2_ref[...], 0.0)
    x_ref[...] = xo
    lg = _mm_bf(xo, lw3_ref[...]) + lb3_ref[...]
    mx = jnp.max(lg, axis=1, keepdims=True)
    z = lg - mx
    out_ref[...] = z - jnp.log(jnp.sum(jnp.exp(z), axis=1, keepdims=True))


def _run_k1(at4, x3d, W1, b1, a1a, a1b):
    fp = jnp.float32
    return pl.pallas_call(
        _k1_body,
        grid=(G,),
        in_specs=[
            pl.BlockSpec((1, P, P), lambda g: (g, 0, 0)),
            pl.BlockSpec((1, P, D), lambda g: (g, 0, 0)),
            pl.BlockSpec((D, NH), lambda g: (0, 0)),
            pl.BlockSpec((1, NH), lambda g: (0, 0)),
            pl.BlockSpec((NH, 1), lambda g: (0, 0)),
            pl.BlockSpec((1, NH), lambda g: (0, 0)),
        ],
        out_specs=[
            pl.BlockSpec((1, P, NH), lambda g: (g, 0, 0)),
            pl.BlockSpec((1, K1, NH), lambda g: (g, 0, 0)),
            pl.BlockSpec((1, K1, K1), lambda g: (g, 0, 0)),
            pl.BlockSpec((1, 1, 2 * NH), lambda g: (g, 0, 0)),
            pl.BlockSpec((1, NH), lambda g: (0, 0)),
            pl.BlockSpec((1, NH), lambda g: (0, 0)),
        ],
        out_shape=[
            jax.ShapeDtypeStruct((G, P, NH), fp),
            jax.ShapeDtypeStruct((G, K1, NH), fp),
            jax.ShapeDtypeStruct((G, K1, K1), fp),
            jax.ShapeDtypeStruct((G, 1, 2 * NH), fp),
            jax.ShapeDtypeStruct((1, NH), fp),
            jax.ShapeDtypeStruct((1, NH), fp),
        ],
    )(at4, x3d, W1, b1, a1a, a1b)


def _run_k2(xs, adj1, r1, bnsum, bnsq, gamma, beta, W2, b2, a2a, a2b, W3, b3):
    fp = jnp.float32
    return pl.pallas_call(
        _k2_body,
        grid=(G,),
        in_specs=[
            pl.BlockSpec((1, K1, NH), lambda g: (g, 0, 0)),
            pl.BlockSpec((1, K1, K1), lambda g: (g, 0, 0)),
            pl.BlockSpec((1, 1, 2 * NH), lambda g: (g, 0, 0)),
            pl.BlockSpec((1, NH), lambda g: (0, 0)),
            pl.BlockSpec((1, NH), lambda g: (0, 0)),
            pl.BlockSpec((1, NH), lambda g: (0, 0)),
            pl.BlockSpec((1, NH), lambda g: (0, 0)),
            pl.BlockSpec((NH, NH), lambda g: (0, 0)),
            pl.BlockSpec((1, NH), lambda g: (0, 0)),
            pl.BlockSpec((NH, 1), lambda g: (0, 0)),
            pl.BlockSpec((1, NH), lambda g: (0, 0)),
            pl.BlockSpec((NH, NH), lambda g: (0, 0)),
            pl.BlockSpec((1, NH), lambda g: (0, 0)),
        ],
        out_specs=[pl.BlockSpec((1, 1, 2 * NH), lambda g: (g, 0, 0))],
        out_shape=[jax.ShapeDtypeStruct((G, 1, 2 * NH), fp)],
    )(xs, adj1, r1, bnsum, bnsq, gamma, beta, W2, b2, a2a, a2b, W3, b3)[0]


def _run_k3(hh, lw1, lb1, lw2, lb2, lw3, lb3):
    fp = jnp.float32
    return pl.pallas_call(
        _k3_body,
        out_shape=[
            jax.ShapeDtypeStruct((G, NH // 2), fp),
            jax.ShapeDtypeStruct((G, NCLS), fp),
        ],
    )(hh, lw1, lb1, lw2, lb2, lw3, lb3)


def kernel(x, edge_index, batch, W1, b1, W2, b2, W3, b3, att1, att2,
           gamma, beta, lw1, lb1, lw2, lb2, lw3, lb3):
    del batch
    fp = jnp.float32
    src = edge_index[0]
    dst = edge_index[1]
    zeros_init = jnp.zeros((_PP,), fp)

    at_flat = _sc_build_at(src, dst, zeros_init)
    at4 = at_flat.reshape(G, P, P)

    x3d = x.reshape(G, P, D)
    b1r = b1.reshape(1, NH)
    a1a = att1[:NH].reshape(NH, 1)
    a1b = att1[NH:].reshape(1, NH)
    a2a = att2[:NH].reshape(NH, 1)
    a2b = att2[NH:].reshape(1, NH)

    x1, xs, adj1, r1, bnsum, bnsq = _run_k1(at4, x3d, W1, b1r, a1a, a1b)
    hh = _run_k2(xs, adj1, r1, bnsum, bnsq, gamma.reshape(1, NH),
                 beta.reshape(1, NH), W2, b2.reshape(1, NH), a2a, a2b,
                 W3, b3.reshape(1, NH))
    x_, out = _run_k3(hh.reshape(G, 2 * NH), lw1, lb1.reshape(1, NH), lw2,
                      lb2.reshape(1, NH // 2), lw3, lb3.reshape(1, NCLS))
    return (x_, out, x1.reshape(N, NH))


# speedup vs baseline: 38.6980x; 1.2351x over previous
import functools
import jax
import jax.numpy as jnp
from jax import lax
from jax.experimental import pallas as pl
from jax.experimental.pallas import tpu as pltpu
from jax.experimental.pallas import tpu_sc as plsc

G, P, D, NH, NCLS = 50, 200, 128, 128, 10
EPG = 6400
LAMB = 1.0
K1 = 100
K2 = 50
N = G * P
E = G * EPG
NPF = N * P


_SC_CORES = 2
_SC_TILES = 16
_NW = _SC_CORES * _SC_TILES
_G2 = G - _NW
_PP = P * P


def _sc_at_body(src_hbm, dst_hbm, zero_hbm, out_hbm, src_v, dst_v, a_v):
    c = lax.axis_index("c")
    s = lax.axis_index("s")
    wid = s * _SC_CORES + c
    two = wid < _G2
    g0 = jnp.where(two, 2 * wid, _G2 + wid)

    ones = jnp.full((16,), 1.0, jnp.float32)

    for i in range(2):
        @pl.when((i == 0) | two)
        def _():
            g = g0 + i
            pltpu.sync_copy(zero_hbm.at[pl.ds(0, _PP)], a_v.at[pl.ds(0, _PP)])
            ebase = g * EPG
            pltpu.sync_copy(src_hbm.at[pl.ds(ebase, EPG)],
                            src_v.at[pl.ds(0, EPG)])
            pltpu.sync_copy(dst_hbm.at[pl.ds(ebase, EPG)],
                            dst_v.at[pl.ds(0, EPG)])

            off = g * (P * (P + 1))

            def _chunk(j, carry):
                sv = src_v[pl.ds(j * 16, 16)]
                dv = dst_v[pl.ds(j * 16, 16)]
                fi = dv * P + sv - off
                plsc.addupdate_scatter(a_v, [fi], ones)
                return carry

            lax.fori_loop(0, EPG // 16, _chunk, 0, unroll=8)

            pltpu.sync_copy(a_v.at[pl.ds(0, _PP)],
                            out_hbm.at[pl.ds(g * _PP, _PP)])


def _sc_build_at(src, dst, zeros_init):
    mesh = plsc.VectorSubcoreMesh(core_axis_name="c", subcore_axis_name="s")
    run = functools.partial(
        pl.kernel,
        mesh=mesh,
        compiler_params=pltpu.CompilerParams(needs_layout_passes=False),
        out_type=jax.ShapeDtypeStruct((NPF,), jnp.float32),
        scratch_types=[
            pltpu.VMEM((EPG,), jnp.int32),
            pltpu.VMEM((EPG,), jnp.int32),
            pltpu.VMEM((_PP,), jnp.float32),
        ],
    )(_sc_at_body)
    return run(src, dst, zeros_init)


def _t_col_to_row(colv, n):
    eye = (lax.broadcasted_iota(jnp.int32, (n, n), 0)
           == lax.broadcasted_iota(jnp.int32, (n, n), 1)).astype(colv.dtype)
    return jnp.sum(eye * colv, axis=0, keepdims=True)


def _perm_matrix(score_col, n, k):
    s_row = _t_col_to_row(score_col, n)
    gt = (s_row > score_col).astype(jnp.float32)
    jlt = (lax.broadcasted_iota(jnp.int32, (n, n), 1)
           < lax.broadcasted_iota(jnp.int32, (n, n), 0))
    eqm = ((s_row == score_col) & jlt).astype(jnp.float32)
    rank = jnp.sum(gt + eqm, axis=1, keepdims=True)
    rank_row = _t_col_to_row(rank, n)
    kio = lax.broadcasted_iota(jnp.int32, (k, n), 0).astype(jnp.float32)
    pm = (kio == rank_row).astype(jnp.float32)
    sv = jnp.sum(pm * s_row, axis=1, keepdims=True)
    return pm, sv


def _dot(a, b):
    return jnp.dot(a, b, preferred_element_type=jnp.float32)


def _dot_nt(a, b):
    return lax.dot_general(a, b, (((1,), (1,)), ((), ())),
                           preferred_element_type=jnp.float32)


def _split3(x):
    b0 = x.astype(jnp.bfloat16)
    r = x - b0.astype(jnp.float32)
    b1 = r.astype(jnp.bfloat16)
    b2 = (r - b1.astype(jnp.float32)).astype(jnp.bfloat16)
    return (b0, b1, b2)


def _mm3s(a_ex, bs):
    ab = a_ex.astype(jnp.bfloat16)
    return (_dot(ab, bs[0]) + _dot(ab, bs[1])) + _dot(ab, bs[2])


def _mm3(a_ex, b):
    return _mm3s(a_ex, _split3(b))


def _mm3_vt(a, b_ex):
    a0, a1, a2 = _split3(a)
    bb = b_ex.astype(jnp.bfloat16)
    return (_dot_nt(a0, bb) + _dot_nt(a1, bb)) + _dot_nt(a2, bb)


def _mm1_nt(a_ex, b_ex):
    return _dot_nt(a_ex.astype(jnp.bfloat16), b_ex.astype(jnp.bfloat16))


def _mm_bf(a, b):
    return jnp.dot(a.astype(jnp.bfloat16), b.astype(jnp.bfloat16),
                   preferred_element_type=jnp.float32)


def _mm_nt_bf(a, b):
    return lax.dot_general(a.astype(jnp.bfloat16), b.astype(jnp.bfloat16),
                           (((1,), (1,)), ((), ())),
                           preferred_element_type=jnp.float32)


def _tc_body(at_ref, x_ref, w1_ref, b1_ref, a1a_ref, a1b_ref, gamma_ref,
             beta_ref, w2_ref, b2_ref, a2a_ref, a2b_ref, w3_ref, b3_ref,
             lw1_ref, lb1_ref, lw2_ref, lb2_ref, lw3_ref, lb3_ref,
             x1_ref, x_out_ref, out_ref,
             x1_s, xs_s, adj1_s, r1_s, hh_s, bnsum_s, bnsq_s):
    ph = pl.program_id(0)
    g = pl.program_id(1)

    @pl.when(ph == 0)
    def _phase0():
        at = at_ref[0]
        x = x_ref[0]

        indeg = jnp.sum(at, axis=1, keepdims=True)
        dinv = lax.rsqrt(indeg + 1.0)
        h = dinv * _mm3(at, x * dinv) + dinv * dinv * x
        x1 = jnp.maximum(_mm_bf(h, w1_ref[...]) + b1_ref[...], 0.0)
        x1_s[g] = x1

        degn = jnp.maximum(indeg, 1.0)
        x1s = _split3(x1)
        neigh = _mm3s(at, x1s) / degn
        score = jnp.sum(jnp.abs(x1 - neigh), axis=1, keepdims=True)

        pm, sv = _perm_matrix(score, P, K1)
        xs = _mm3s(pm, x1s) * jnp.tanh(sv)
        xs_s[g] = xs

        u = _mm1_nt(pm, at)
        asel = _mm1_nt(u, pm)

        s_i = _mm_bf(xs, a1a_ref[...])
        s_j = _mm_nt_bf(a1b_ref[...], xs)
        e1 = jnp.maximum(s_i + s_j, 0.0) + LAMB * asel
        m = jnp.max(e1, axis=1, keepdims=True)
        ex = jnp.exp(e1 - m)
        adj1_s[g] = ex / jnp.sum(ex, axis=1, keepdims=True)

        r1_s[g, :, 0:NH] = jnp.max(xs, axis=0, keepdims=True)
        r1_s[g, :, NH:2 * NH] = jnp.mean(xs, axis=0, keepdims=True)

        @pl.when(g == 0)
        def _():
            bnsum_s[...] = jnp.zeros_like(bnsum_s)
            bnsq_s[...] = jnp.zeros_like(bnsq_s)

        bnsum_s[...] += jnp.sum(xs, axis=0, keepdims=True)
        bnsq_s[...] += jnp.sum(xs * xs, axis=0, keepdims=True)

    @pl.when(ph == 1)
    def _phase1():
        x1_ref[0] = x1_s[g]
        xs = xs_s[g]
        adj1 = adj1_s[g]
        inv_n = 1.0 / (G * K1)
        mu = bnsum_s[...] * inv_n
        var = bnsq_s[...] * inv_n - mu * mu
        xb = (xs - mu) * lax.rsqrt(var + 1e-5) * gamma_ref[...] + beta_ref[...]

        x2 = jnp.maximum(_mm_bf(_mm_bf(adj1, xb), w2_ref[...]) + b2_ref[...],
                         0.0)
        neigh2 = _mm_bf(adj1, x2)
        score2 = jnp.sum(jnp.abs(x2 - neigh2), axis=1, keepdims=True)

        pm2, sv2 = _perm_matrix(score2, K1, K2)
        xs2 = _mm3(pm2, x2) * jnp.tanh(sv2)

        u2 = _mm3(pm2, adj1)
        a1sel = _mm3_vt(u2, pm2)

        t_i = _mm_bf(xs2, a2a_ref[...])
        t_j = _mm_nt_bf(a2b_ref[...], xs2)
        e2 = jnp.maximum(t_i + t_j, 0.0) + LAMB * a1sel
        m2 = jnp.max(e2, axis=1, keepdims=True)
        ex2 = jnp.exp(e2 - m2)
        adj2 = ex2 / jnp.sum(ex2, axis=1, keepdims=True)

        x3 = jnp.maximum(_mm_bf(_mm_bf(adj2, xs2), w3_ref[...]) + b3_ref[...],
                         0.0)

        r2 = jnp.concatenate([jnp.max(xs2, axis=0, keepdims=True),
                              jnp.mean(xs2, axis=0, keepdims=True)], axis=1)
        r3 = jnp.concatenate([jnp.max(x3, axis=0, keepdims=True),
                              jnp.mean(x3, axis=0, keepdims=True)], axis=1)
        hh_s[g] = (jnp.maximum(r1_s[g], 0.0) + jnp.maximum(r2, 0.0)
                   + jnp.maximum(r3, 0.0))

        @pl.when(g == G - 1)
        def _head():
            hh_all = jnp.squeeze(hh_s[...], axis=1)
            hh = jnp.maximum(_mm_bf(hh_all, lw1_ref[...]) + lb1_ref[...],
                             0.0)
            xo = jnp.maximum(_mm_bf(hh, lw2_ref[...]) + lb2_ref[...], 0.0)
            x_out_ref[...] = xo
            lg = _mm_bf(xo, lw3_ref[...]) + lb3_ref[...]
            z = lg - jnp.max(lg, axis=1, keepdims=True)
            out_ref[...] = z - jnp.log(jnp.sum(jnp.exp(z), axis=1,
                                               keepdims=True))


def _run_tc(at4, x3d, W1, b1, a1a, a1b, gamma, beta, W2, b2, a2a, a2b, W3, b3,
            lw1, lb1, lw2, lb2, lw3, lb3):
    fp = jnp.float32
    full2 = lambda ph, g: (0, 0)
    return pl.pallas_call(
        _tc_body,
        grid=(2, G),
        in_specs=[
            pl.BlockSpec((1, P, P), lambda ph, g: ((1 - ph) * g, 0, 0)),
            pl.BlockSpec((1, P, D), lambda ph, g: ((1 - ph) * g, 0, 0)),
            pl.BlockSpec((D, NH), full2),
            pl.BlockSpec((1, NH), full2),
            pl.BlockSpec((NH, 1), full2),
            pl.BlockSpec((1, NH), full2),
            pl.BlockSpec((1, NH), full2),
            pl.BlockSpec((1, NH), full2),
            pl.BlockSpec((NH, NH), full2),
            pl.BlockSpec((1, NH), full2),
            pl.BlockSpec((NH, 1), full2),
            pl.BlockSpec((1, NH), full2),
            pl.BlockSpec((NH, NH), full2),
            pl.BlockSpec((1, NH), full2),
            pl.BlockSpec((2 * NH, NH), full2),
            pl.BlockSpec((1, NH), full2),
            pl.BlockSpec((NH, NH // 2), full2),
            pl.BlockSpec((1, NH // 2), full2),
            pl.BlockSpec((NH // 2, NCLS), full2),
            pl.BlockSpec((1, NCLS), full2),
        ],
        out_specs=[
            pl.BlockSpec((1, P, NH), lambda ph, g: (ph * g, 0, 0)),
            pl.BlockSpec((G, NH // 2), full2),
            pl.BlockSpec((G, NCLS), full2),
        ],
        out_shape=[
            jax.ShapeDtypeStruct((G, P, NH), fp),
            jax.ShapeDtypeStruct((G, NH // 2), fp),
            jax.ShapeDtypeStruct((G, NCLS), fp),
        ],
        scratch_shapes=[
            pltpu.VMEM((G, P, NH), fp),
            pltpu.VMEM((G, K1, NH), fp),
            pltpu.VMEM((G, K1, K1), fp),
            pltpu.VMEM((G, 1, 2 * NH), fp),
            pltpu.VMEM((G, 1, 2 * NH), fp),
            pltpu.VMEM((1, NH), fp),
            pltpu.VMEM((1, NH), fp),
        ],
    )(at4, x3d, W1, b1, a1a, a1b, gamma, beta, W2, b2, a2a, a2b, W3, b3,
      lw1, lb1, lw2, lb2, lw3, lb3)


def kernel(x, edge_index, batch, W1, b1, W2, b2, W3, b3, att1, att2,
           gamma, beta, lw1, lb1, lw2, lb2, lw3, lb3):
    del batch
    fp = jnp.float32
    src = edge_index[0]
    dst = edge_index[1]
    zeros_init = jnp.zeros((_PP,), fp)

    at_flat = _sc_build_at(src, dst, zeros_init)
    at4 = at_flat.reshape(G, P, P)

    x3d = x.reshape(G, P, D)
    b1r = b1.reshape(1, NH)
    a1a = att1[:NH].reshape(NH, 1)
    a1b = att1[NH:].reshape(1, NH)
    a2a = att2[:NH].reshape(NH, 1)
    a2b = att2[NH:].reshape(1, NH)

    x1, x_, out = _run_tc(at4, x3d, W1, b1r, a1a, a1b, gamma.reshape(1, NH),
                          beta.reshape(1, NH), W2, b2.reshape(1, NH), a2a,
                          a2b, W3, b3.reshape(1, NH), lw1,
                          lb1.reshape(1, NH), lw2, lb2.reshape(1, NH // 2),
                          lw3, lb3.reshape(1, NCLS))
    return (x_, out, x1.reshape(N, NH))


# speedup vs baseline: 43.1100x; 1.1140x over previous
import functools
import jax
import jax.numpy as jnp
from jax import lax
from jax.experimental import pallas as pl
from jax.experimental.pallas import tpu as pltpu
from jax.experimental.pallas import tpu_sc as plsc

G, P, D, NH, NCLS = 50, 200, 128, 128, 10
EPG = 6400
LAMB = 1.0
K1 = 100
K2 = 50
N = G * P
E = G * EPG
NPF = N * P


_SC_CORES = 2
_SC_TILES = 16
_NW = _SC_CORES * _SC_TILES
_G2 = G - _NW
_PP = P * P


def _sc_at_body(src_hbm, dst_hbm, zero_hbm, out_hbm, src_v, dst_v, a_v):
    c = lax.axis_index("c")
    s = lax.axis_index("s")
    wid = s * _SC_CORES + c
    two = wid < _G2
    g0 = jnp.where(two, 2 * wid, _G2 + wid)

    ones = jnp.full((16,), 1.0, jnp.float32)

    for i in range(2):
        @pl.when((i == 0) | two)
        def _():
            g = g0 + i
            pltpu.sync_copy(zero_hbm.at[pl.ds(0, _PP)], a_v.at[pl.ds(0, _PP)])
            ebase = g * EPG
            pltpu.sync_copy(src_hbm.at[pl.ds(ebase, EPG)],
                            src_v.at[pl.ds(0, EPG)])
            pltpu.sync_copy(dst_hbm.at[pl.ds(ebase, EPG)],
                            dst_v.at[pl.ds(0, EPG)])

            off = g * (P * (P + 1))

            def _chunk(j, carry):
                sv = src_v[pl.ds(j * 16, 16)]
                dv = dst_v[pl.ds(j * 16, 16)]
                fi = dv * P + sv - off
                plsc.addupdate_scatter(a_v, [fi], ones)
                return carry

            lax.fori_loop(0, EPG // 16, _chunk, 0, unroll=8)

            pltpu.sync_copy(a_v.at[pl.ds(0, _PP)],
                            out_hbm.at[pl.ds(g * _PP, _PP)])


def _sc_build_at(src, dst, zeros_init):
    mesh = plsc.VectorSubcoreMesh(core_axis_name="c", subcore_axis_name="s")
    run = functools.partial(
        pl.kernel,
        mesh=mesh,
        compiler_params=pltpu.CompilerParams(needs_layout_passes=False),
        out_type=jax.ShapeDtypeStruct((NPF,), jnp.float32),
        scratch_types=[
            pltpu.VMEM((EPG,), jnp.int32),
            pltpu.VMEM((EPG,), jnp.int32),
            pltpu.VMEM((_PP,), jnp.float32),
        ],
    )(_sc_at_body)
    return run(src, dst, zeros_init)


def _t_col_to_row(colv, n):
    eye = (lax.broadcasted_iota(jnp.int32, (n, n), 0)
           == lax.broadcasted_iota(jnp.int32, (n, n), 1)).astype(colv.dtype)
    return jnp.sum(eye * colv, axis=0, keepdims=True)


def _perm_matrix(score_col, n, k):
    s_row = _t_col_to_row(score_col, n)
    gt = (s_row > score_col).astype(jnp.float32)
    jlt = (lax.broadcasted_iota(jnp.int32, (n, n), 1)
           < lax.broadcasted_iota(jnp.int32, (n, n), 0))
    eqm = ((s_row == score_col) & jlt).astype(jnp.float32)
    rank = jnp.sum(gt + eqm, axis=1, keepdims=True)
    rank_row = _t_col_to_row(rank, n)
    kio = lax.broadcasted_iota(jnp.int32, (k, n), 0).astype(jnp.float32)
    pm = (kio == rank_row).astype(jnp.float32)
    sv = jnp.sum(pm * s_row, axis=1, keepdims=True)
    return pm, sv


def _dot(a, b):
    return jnp.dot(a, b, preferred_element_type=jnp.float32)


def _dot_nt(a, b):
    return lax.dot_general(a, b, (((1,), (1,)), ((), ())),
                           preferred_element_type=jnp.float32)


def _split3(x):
    b0 = x.astype(jnp.bfloat16)
    r = x - b0.astype(jnp.float32)
    b1 = r.astype(jnp.bfloat16)
    b2 = (r - b1.astype(jnp.float32)).astype(jnp.bfloat16)
    return (b0, b1, b2)


def _mm3s(a_ex, bs):
    ab = a_ex.astype(jnp.bfloat16)
    return (_dot(ab, bs[0]) + _dot(ab, bs[1])) + _dot(ab, bs[2])


def _mm3(a_ex, b):
    return _mm3s(a_ex, _split3(b))


def _mm3_vt(a, b_ex):
    a0, a1, a2 = _split3(a)
    bb = b_ex.astype(jnp.bfloat16)
    return (_dot_nt(a0, bb) + _dot_nt(a1, bb)) + _dot_nt(a2, bb)


def _mm1_nt(a_ex, b_ex):
    return _dot_nt(a_ex.astype(jnp.bfloat16), b_ex.astype(jnp.bfloat16))


def _mm_bf(a, b):
    return jnp.dot(a.astype(jnp.bfloat16), b.astype(jnp.bfloat16),
                   preferred_element_type=jnp.float32)


def _mm_nt_bf(a, b):
    return lax.dot_general(a.astype(jnp.bfloat16), b.astype(jnp.bfloat16),
                           (((1,), (1,)), ((), ())),
                           preferred_element_type=jnp.float32)


def _tc_body(at_ref, x_ref, w1_ref, b1_ref, a1a_ref, a1b_ref, gamma_ref,
             beta_ref, w2_ref, b2_ref, a2a_ref, a2b_ref, w3_ref, b3_ref,
             lw1_ref, lb1_ref, lw2_ref, lb2_ref, lw3_ref, lb3_ref,
             x1_ref, x_out_ref, out_ref,
             x1_s, xs_s, adj1_s, r1_s, hh_s, bnsum_s, bnsq_s):
    ph = pl.program_id(0)
    gg = pl.program_id(1)

    @pl.when(ph == 0)
    def _phase0():
        @pl.when(gg == 0)
        def _():
            bnsum_s[...] = jnp.zeros_like(bnsum_s)
            bnsq_s[...] = jnp.zeros_like(bnsq_s)

        for i in range(2):
            g = 2 * gg + i
            at = at_ref[i]
            x = x_ref[i]

            indeg = jnp.sum(at, axis=1, keepdims=True)
            dinv = lax.rsqrt(indeg + 1.0)
            h = dinv * _mm3(at, x * dinv) + dinv * dinv * x
            x1 = jnp.maximum(_mm_bf(h, w1_ref[...]) + b1_ref[...], 0.0)
            x1_s[g] = x1

            degn = jnp.maximum(indeg, 1.0)
            x1s = _split3(x1)
            neigh = _mm3s(at, x1s) / degn
            score = jnp.sum(jnp.abs(x1 - neigh), axis=1, keepdims=True)

            pm, sv = _perm_matrix(score, P, K1)
            xs = _mm3s(pm, x1s) * jnp.tanh(sv)
            xs_s[g] = xs

            u = _mm1_nt(pm, at)
            asel = _mm1_nt(u, pm)

            s_i = _mm_bf(xs, a1a_ref[...])
            s_j = _mm_nt_bf(a1b_ref[...], xs)
            e1 = jnp.maximum(s_i + s_j, 0.0) + LAMB * asel
            m = jnp.max(e1, axis=1, keepdims=True)
            ex = jnp.exp(e1 - m)
            adj1_s[g] = ex / jnp.sum(ex, axis=1, keepdims=True)

            r1_s[g, :, 0:NH] = jnp.max(xs, axis=0, keepdims=True)
            r1_s[g, :, NH:2 * NH] = jnp.mean(xs, axis=0, keepdims=True)

            bnsum_s[...] += jnp.sum(xs, axis=0, keepdims=True)
            bnsq_s[...] += jnp.sum(xs * xs, axis=0, keepdims=True)

    @pl.when(ph == 1)
    def _phase1():
        inv_n = 1.0 / (G * K1)
        mu = bnsum_s[...] * inv_n
        var = bnsq_s[...] * inv_n - mu * mu
        rstd = lax.rsqrt(var + 1e-5)

        for i in range(2):
            g = 2 * gg + i
            x1_ref[i] = x1_s[g]
            xs = xs_s[g]
            adj1 = adj1_s[g]
            xb = (xs - mu) * rstd * gamma_ref[...] + beta_ref[...]

            x2 = jnp.maximum(
                _mm_bf(_mm_bf(adj1, xb), w2_ref[...]) + b2_ref[...], 0.0)
            neigh2 = _mm_bf(adj1, x2)
            score2 = jnp.sum(jnp.abs(x2 - neigh2), axis=1, keepdims=True)

            pm2, sv2 = _perm_matrix(score2, K1, K2)
            xs2 = _mm3(pm2, x2) * jnp.tanh(sv2)

            u2 = _mm3(pm2, adj1)
            a1sel = _mm3_vt(u2, pm2)

            t_i = _mm_bf(xs2, a2a_ref[...])
            t_j = _mm_nt_bf(a2b_ref[...], xs2)
            e2 = jnp.maximum(t_i + t_j, 0.0) + LAMB * a1sel
            m2 = jnp.max(e2, axis=1, keepdims=True)
            ex2 = jnp.exp(e2 - m2)
            adj2 = ex2 / jnp.sum(ex2, axis=1, keepdims=True)

            x3 = jnp.maximum(
                _mm_bf(_mm_bf(adj2, xs2), w3_ref[...]) + b3_ref[...], 0.0)

            r2 = jnp.concatenate([jnp.max(xs2, axis=0, keepdims=True),
                                  jnp.mean(xs2, axis=0, keepdims=True)],
                                 axis=1)
            r3 = jnp.concatenate([jnp.max(x3, axis=0, keepdims=True),
                                  jnp.mean(x3, axis=0, keepdims=True)],
                                 axis=1)
            hh_s[g] = (jnp.maximum(r1_s[g], 0.0) + jnp.maximum(r2, 0.0)
                       + jnp.maximum(r3, 0.0))

        @pl.when(gg == G // 2 - 1)
        def _head():
            hh_all = jnp.squeeze(hh_s[...], axis=1)
            hh = jnp.maximum(_mm_bf(hh_all, lw1_ref[...]) + lb1_ref[...],
                             0.0)
            xo = jnp.maximum(_mm_bf(hh, lw2_ref[...]) + lb2_ref[...], 0.0)
            x_out_ref[...] = xo
            lg = _mm_bf(xo, lw3_ref[...]) + lb3_ref[...]
            z = lg - jnp.max(lg, axis=1, keepdims=True)
            out_ref[...] = z - jnp.log(jnp.sum(jnp.exp(z), axis=1,
                                               keepdims=True))


def _run_tc(at4, x3d, W1, b1, a1a, a1b, gamma, beta, W2, b2, a2a, a2b, W3, b3,
            lw1, lb1, lw2, lb2, lw3, lb3):
    fp = jnp.float32
    full2 = lambda ph, g: (0, 0)
    return pl.pallas_call(
        _tc_body,
        grid=(2, G // 2),
        in_specs=[
            pl.BlockSpec((2, P, P), lambda ph, g: ((1 - ph) * g, 0, 0)),
            pl.BlockSpec((2, P, D), lambda ph, g: ((1 - ph) * g, 0, 0)),
            pl.BlockSpec((D, NH), full2),
            pl.BlockSpec((1, NH), full2),
            pl.BlockSpec((NH, 1), full2),
            pl.BlockSpec((1, NH), full2),
            pl.BlockSpec((1, NH), full2),
            pl.BlockSpec((1, NH), full2),
            pl.BlockSpec((NH, NH), full2),
            pl.BlockSpec((1, NH), full2),
            pl.BlockSpec((NH, 1), full2),
            pl.BlockSpec((1, NH), full2),
            pl.BlockSpec((NH, NH), full2),
            pl.BlockSpec((1, NH), full2),
            pl.BlockSpec((2 * NH, NH), full2),
            pl.BlockSpec((1, NH), full2),
            pl.BlockSpec((NH, NH // 2), full2),
            pl.BlockSpec((1, NH // 2), full2),
            pl.BlockSpec((NH // 2, NCLS), full2),
            pl.BlockSpec((1, NCLS), full2),
        ],
        out_specs=[
            pl.BlockSpec((2, P, NH), lambda ph, g: (ph * g, 0, 0)),
            pl.BlockSpec((G, NH // 2), full2),
            pl.BlockSpec((G, NCLS), full2),
        ],
        out_shape=[
            jax.ShapeDtypeStruct((G, P, NH), fp),
            jax.ShapeDtypeStruct((G, NH // 2), fp),
            jax.ShapeDtypeStruct((G, NCLS), fp),
        ],
        scratch_shapes=[
            pltpu.VMEM((G, P, NH), fp),
            pltpu.VMEM((G, K1, NH), fp),
            pltpu.VMEM((G, K1, K1), fp),
            pltpu.VMEM((G, 1, 2 * NH), fp),
            pltpu.VMEM((G, 1, 2 * NH), fp),
            pltpu.VMEM((1, NH), fp),
            pltpu.VMEM((1, NH), fp),
        ],
    )(at4, x3d, W1, b1, a1a, a1b, gamma, beta, W2, b2, a2a, a2b, W3, b3,
      lw1, lb1, lw2, lb2, lw3, lb3)


def kernel(x, edge_index, batch, W1, b1, W2, b2, W3, b3, att1, att2,
           gamma, beta, lw1, lb1, lw2, lb2, lw3, lb3):
    del batch
    fp = jnp.float32
    src = edge_index[0]
    dst = edge_index[1]
    zeros_init = jnp.zeros((_PP,), fp)

    at_flat = _sc_build_at(src, dst, zeros_init)
    at4 = at_flat.reshape(G, P, P)

    x3d = x.reshape(G, P, D)
    b1r = b1.reshape(1, NH)
    a1a = att1[:NH].reshape(NH, 1)
    a1b = att1[NH:].reshape(1, NH)
    a2a = att2[:NH].reshape(NH, 1)
    a2b = att2[NH:].reshape(1, NH)

    x1, x_, out = _run_tc(at4, x3d, W1, b1r, a1a, a1b, gamma.reshape(1, NH),
                          beta.reshape(1, NH), W2, b2.reshape(1, NH), a2a,
                          a2b, W3, b3.reshape(1, NH), lw1,
                          lb1.reshape(1, NH), lw2, lb2.reshape(1, NH // 2),
                          lw3, lb3.reshape(1, NCLS))
    return (x_, out, x1.reshape(N, NH))
